# baseline (speedup 1.0000x reference)
import jax
import jax.numpy as jnp
from jax.experimental import pallas as pl
from jax.experimental.pallas import tpu as pltpu

_B, _CIN, _N, _T = 8, 32, 1024, 12
_COUT = 64
_NPIECES = 5
_NT = 512
_M = _B * _T * _CIN
_NGRP = _M // _CIN
_R = 8


def _main_kernel(x_ref, s_ref, w_ref, bias_ref, y_ref, ab_s, a2_s):
    r = pl.program_id(0)
    j = pl.program_id(1)

    @pl.when((j == 0) & (r == 0))
    def _prep():
        a0 = s_ref[0].astype(jnp.bfloat16)
        a1 = s_ref[1].astype(jnp.bfloat16)
        ab_s[0] = a0
        ab_s[1] = a1
        a2_s[0] = jnp.dot(a0, a0, preferred_element_type=jnp.float32).astype(
            jnp.bfloat16)
        a2_s[1] = jnp.dot(a1, a1, preferred_element_type=jnp.float32).astype(
            jnp.bfloat16)

    X = x_ref[...]
    sl = pl.ds(j * _NT, _NT)
    Mcat = jnp.concatenate(
        [ab_s[0, :, sl], a2_s[0, :, sl], ab_s[1, :, sl], a2_s[1, :, sl]],
        axis=1)
    H = jnp.dot(X, Mcat, preferred_element_type=jnp.float32).astype(
        jnp.bfloat16)
    pieces = [x_ref[:, sl]]
    for k in range(4):
        pieces.append(H[:, k * _NT:(k + 1) * _NT])
    Hcat = jnp.concatenate(
        [p.reshape(_NGRP // _R, _CIN, _NT) for p in pieces], axis=1)
    Wb = jnp.broadcast_to(w_ref[...][None],
                          (_NGRP // _R, _COUT, _NPIECES * _CIN))
    acc = jax.lax.dot_general(
        Wb, Hcat, (((2,), (1,)), ((0,), (0,))),
        preferred_element_type=jnp.float32)
    acc = acc + bias_ref[...][None, :, :]
    y_ref[...] = acc.reshape(_B * _T * _COUT // _R, _NT)


def kernel(x, supports, W, b):
    B, C, N, T = x.shape
    Xr = x.transpose(0, 3, 1, 2).reshape(B * T * C, N).astype(jnp.bfloat16)
    Wb16 = W.astype(jnp.bfloat16)
    bias = b.reshape(_COUT, 1)

    Y = pl.pallas_call(
        _main_kernel,
        grid=(_R, N // _NT),
        in_specs=[
            pl.BlockSpec((B * T * C // _R, N), lambda r, j: (r, 0)),
            pl.BlockSpec((2, N, N), lambda r, j: (0, 0, 0)),
            pl.BlockSpec((_COUT, _NPIECES * _CIN), lambda r, j: (0, 0)),
            pl.BlockSpec((_COUT, 1), lambda r, j: (0, 0)),
        ],
        out_specs=pl.BlockSpec((B * T * _COUT // _R, _NT), lambda r, j: (r, j)),
        out_shape=jax.ShapeDtypeStruct((B * T * _COUT, N), jnp.float32),
        scratch_shapes=[
            pltpu.VMEM((2, N, N), jnp.bfloat16),
            pltpu.VMEM((2, N, N), jnp.bfloat16),
        ],
    )(Xr, supports, Wb16, bias)

    return Y.reshape(B, T, _COUT, N).transpose(0, 2, 3, 1)


# speedup vs baseline: 1.0626x; 1.0626x over previous
import jax
import jax.numpy as jnp
from jax.experimental import pallas as pl
from jax.experimental.pallas import tpu as pltpu

_B, _CIN, _N, _T = 8, 32, 1024, 12
_COUT = 64
_NPIECES = 5
_NT = 1024
_M = _B * _T * _CIN
_NGRP = _M // _CIN
_R = 4


def _main_kernel(x_ref, s_ref, w_ref, bias_ref, y_ref, ab_s, a2_s):
    r = pl.program_id(0)
    j = pl.program_id(1)

    @pl.when((j == 0) & (r == 0))
    def _prep():
        a0 = s_ref[0].astype(jnp.bfloat16)
        a1 = s_ref[1].astype(jnp.bfloat16)
        ab_s[0] = a0
        ab_s[1] = a1
        a2_s[0] = jnp.dot(a0, a0, preferred_element_type=jnp.float32).astype(
            jnp.bfloat16)
        a2_s[1] = jnp.dot(a1, a1, preferred_element_type=jnp.float32).astype(
            jnp.bfloat16)

    X = x_ref[...]
    sl = pl.ds(j * _NT, _NT)
    Mcat = jnp.concatenate(
        [ab_s[0, :, sl], a2_s[0, :, sl], ab_s[1, :, sl], a2_s[1, :, sl]],
        axis=1)
    H = jnp.dot(X, Mcat, preferred_element_type=jnp.float32).astype(
        jnp.bfloat16)
    pieces = [x_ref[:, sl]]
    for k in range(4):
        pieces.append(H[:, k * _NT:(k + 1) * _NT])
    Hcat = jnp.concatenate(
        [p.reshape(_NGRP // _R, _CIN, _NT) for p in pieces], axis=1)
    Wb = jnp.broadcast_to(w_ref[...][None],
                          (_NGRP // _R, _COUT, _NPIECES * _CIN))
    acc = jax.lax.dot_general(
        Wb, Hcat, (((2,), (1,)), ((0,), (0,))),
        preferred_element_type=jnp.float32)
    acc = acc + bias_ref[...][None, :, :]
    y_ref[...] = acc.reshape(_B * _T * _COUT // _R, _NT)


def kernel(x, supports, W, b):
    B, C, N, T = x.shape
    Xr = x.transpose(0, 3, 1, 2).reshape(B * T * C, N).astype(jnp.bfloat16)
    Wb16 = W.astype(jnp.bfloat16)
    bias = b.reshape(_COUT, 1)

    Y = pl.pallas_call(
        _main_kernel,
        grid=(_R, N // _NT),
        in_specs=[
            pl.BlockSpec((B * T * C // _R, N), lambda r, j: (r, 0)),
            pl.BlockSpec((2, N, N), lambda r, j: (0, 0, 0)),
            pl.BlockSpec((_COUT, _NPIECES * _CIN), lambda r, j: (0, 0)),
            pl.BlockSpec((_COUT, 1), lambda r, j: (0, 0)),
        ],
        out_specs=pl.BlockSpec((B * T * _COUT // _R, _NT), lambda r, j: (r, j)),
        out_shape=jax.ShapeDtypeStruct((B * T * _COUT, N), jnp.float32),
        scratch_shapes=[
            pltpu.VMEM((2, N, N), jnp.bfloat16),
            pltpu.VMEM((2, N, N), jnp.bfloat16),
        ],
    )(Xr, supports, Wb16, bias)

    return Y.reshape(B, T, _COUT, N).transpose(0, 2, 3, 1)


# speedup vs baseline: 1.0668x; 1.0039x over previous
import jax
import jax.numpy as jnp
from jax.experimental import pallas as pl
from jax.experimental.pallas import tpu as pltpu

_B, _CIN, _N, _T = 8, 32, 1024, 12
_COUT = 64
_NPIECES = 5
_NT = 1024
_M = _B * _T * _CIN
_NGRP = _M // _CIN
_R = 8


def _main_kernel(x_ref, s_ref, w_ref, bias_ref, y_ref, ab_s, a2_s):
    r = pl.program_id(0)
    j = pl.program_id(1)

    @pl.when((j == 0) & (r == 0))
    def _prep():
        a0 = s_ref[0].astype(jnp.bfloat16)
        a1 = s_ref[1].astype(jnp.bfloat16)
        ab_s[0] = a0
        ab_s[1] = a1
        a2_s[0] = jnp.dot(a0, a0, preferred_element_type=jnp.float32).astype(
            jnp.bfloat16)
        a2_s[1] = jnp.dot(a1, a1, preferred_element_type=jnp.float32).astype(
            jnp.bfloat16)

    X = x_ref[...]
    sl = pl.ds(j * _NT, _NT)
    Mcat = jnp.concatenate(
        [ab_s[0, :, sl], a2_s[0, :, sl], ab_s[1, :, sl], a2_s[1, :, sl]],
        axis=1)
    H = jnp.dot(X, Mcat, preferred_element_type=jnp.float32).astype(
        jnp.bfloat16)
    pieces = [x_ref[:, sl]]
    for k in range(4):
        pieces.append(H[:, k * _NT:(k + 1) * _NT])
    Hcat = jnp.concatenate(
        [p.reshape(_NGRP // _R, _CIN, _NT) for p in pieces], axis=1)
    Wb = jnp.broadcast_to(w_ref[...][None],
                          (_NGRP // _R, _COUT, _NPIECES * _CIN))
    acc = jax.lax.dot_general(
        Wb, Hcat, (((2,), (1,)), ((0,), (0,))),
        preferred_element_type=jnp.float32)
    acc = acc + bias_ref[...][None, :, :]
    y_ref[...] = acc.reshape(_B * _T * _COUT // _R, _NT)


def kernel(x, supports, W, b):
    B, C, N, T = x.shape
    Xr = x.transpose(0, 3, 1, 2).reshape(B * T * C, N).astype(jnp.bfloat16)
    Wb16 = W.astype(jnp.bfloat16)
    bias = b.reshape(_COUT, 1)

    Y = pl.pallas_call(
        _main_kernel,
        grid=(_R, N // _NT),
        in_specs=[
            pl.BlockSpec((B * T * C // _R, N), lambda r, j: (r, 0)),
            pl.BlockSpec((2, N, N), lambda r, j: (0, 0, 0)),
            pl.BlockSpec((_COUT, _NPIECES * _CIN), lambda r, j: (0, 0)),
            pl.BlockSpec((_COUT, 1), lambda r, j: (0, 0)),
        ],
        out_specs=pl.BlockSpec((B * T * _COUT // _R, _NT), lambda r, j: (r, j)),
        out_shape=jax.ShapeDtypeStruct((B * T * _COUT, N), jnp.float32),
        scratch_shapes=[
            pltpu.VMEM((2, N, N), jnp.bfloat16),
            pltpu.VMEM((2, N, N), jnp.bfloat16),
        ],
    )(Xr, supports, Wb16, bias)

    return Y.reshape(B, T, _COUT, N).transpose(0, 2, 3, 1)
